# Initial kernel scaffold; baseline (speedup 1.0000x reference)
import jax
import jax.numpy as jnp
from jax.experimental import pallas as pl


def kernel(hidden_states, gate_w, Wg, Wu, Wd, Sg, Su, Sd, seg_w):
    raise NotImplementedError("write your pallas kernel here")


# speedup vs baseline: 1.2153x; 1.2153x over previous
import functools

import jax
import jax.numpy as jnp
from jax.experimental import pallas as pl
from jax.experimental.pallas import tpu as pltpu

HIDDEN = 1024
MOE_FF = 512
SHARED_FF = 1024
E = 8
TOP_K = 2
T = 2048
TBS = 512
TB = 256
NB = T // TB


def _dot_t(a, b, prec=jnp.float32):
    return jax.lax.dot_general(a, b, (((1,), (1,)), ((), ())),
                               preferred_element_type=prec)


def _shared_body(x_ref, sg_ref, su_ref, sd_ref, segw_ref, out_ref):
    x = x_ref[...]
    xb = x.astype(jnp.bfloat16)
    g = _dot_t(xb, sg_ref[...])
    u = _dot_t(xb, su_ref[...])
    h = (g * jax.nn.sigmoid(g) * u).astype(jnp.bfloat16)
    sh = _dot_t(h, sd_ref[...])
    sgate = jax.nn.sigmoid(_dot_t(x, segw_ref[...]))
    out_ref[...] = sgate * sh


def _moe_body(x_ref, gw_ref, wg_ref, wu_ref, wd_ref, part_ref, out_ref,
              acc_ref, dw_ref):
    e = pl.program_id(0)
    b = pl.program_id(1)

    @pl.when(e == 0)
    def _router():
        logits = _dot_t(x_ref[...], gw_ref[...])
        iota = jax.lax.broadcasted_iota(jnp.int32, (TB, E), 1)
        m1 = jnp.max(logits, axis=1, keepdims=True)
        i1 = jnp.min(jnp.where(logits == m1, iota, E), axis=1, keepdims=True)
        sel1 = iota == i1
        masked = jnp.where(sel1, -jnp.inf, logits)
        m2 = jnp.max(masked, axis=1, keepdims=True)
        i2 = jnp.min(jnp.where(masked == m2, iota, E), axis=1, keepdims=True)
        sel2 = iota == i2
        d = jnp.exp(m2 - m1)
        w1 = 1.0 / (1.0 + d)
        w2 = 1.0 - w1
        dw = jnp.where(sel1, w1, 0.0) + jnp.where(sel2, w2, 0.0)
        dw_ref[pl.ds(b * TB, TB), :] = dw

    xb = x_ref[...].astype(jnp.bfloat16)
    g = _dot_t(xb, wg_ref[0])
    u = _dot_t(xb, wu_ref[0])
    h = (g * jax.nn.sigmoid(g) * u).astype(jnp.bfloat16)
    eo = _dot_t(h, wd_ref[0])
    onehot = (jax.lax.broadcasted_iota(jnp.int32, (E, 1), 0) == e
              ).astype(jnp.float32)
    scale = jax.lax.dot_general(dw_ref[pl.ds(b * TB, TB), :], onehot,
                                (((1,), (0,)), ((), ())),
                                preferred_element_type=jnp.float32)
    contrib = scale * eo
    row = pl.ds(b * TB, TB)

    @pl.when(e == 0)
    def _init():
        acc_ref[row, :] = part_ref[...] + contrib

    @pl.when(e > 0)
    def _accum():
        acc_ref[row, :] = acc_ref[row, :] + contrib

    @pl.when(e == E - 1)
    def _write():
        out_ref[...] = acc_ref[row, :]


@jax.jit
def kernel(hidden_states, gate_w, Wg, Wu, Wd, Sg, Su, Sd, seg_w):
    bsz, s, d = hidden_states.shape
    x = hidden_states.reshape(bsz * s, d)

    partial = pl.pallas_call(
        _shared_body,
        grid=(T // TBS,),
        in_specs=[
            pl.BlockSpec((TBS, HIDDEN), lambda i: (i, 0)),
            pl.BlockSpec((SHARED_FF, HIDDEN), lambda i: (0, 0)),
            pl.BlockSpec((SHARED_FF, HIDDEN), lambda i: (0, 0)),
            pl.BlockSpec((HIDDEN, SHARED_FF), lambda i: (0, 0)),
            pl.BlockSpec((1, HIDDEN), lambda i: (0, 0)),
        ],
        out_specs=pl.BlockSpec((TBS, HIDDEN), lambda i: (i, 0)),
        out_shape=jax.ShapeDtypeStruct((T, HIDDEN), jnp.float32),
    )(x, Sg.astype(jnp.bfloat16), Su.astype(jnp.bfloat16),
      Sd.astype(jnp.bfloat16), seg_w)

    out = pl.pallas_call(
        _moe_body,
        grid=(E, NB),
        in_specs=[
            pl.BlockSpec((TB, HIDDEN), lambda e, b: (b, 0)),
            pl.BlockSpec((E, HIDDEN), lambda e, b: (0, 0)),
            pl.BlockSpec((1, MOE_FF, HIDDEN), lambda e, b: (e, 0, 0)),
            pl.BlockSpec((1, MOE_FF, HIDDEN), lambda e, b: (e, 0, 0)),
            pl.BlockSpec((1, HIDDEN, MOE_FF), lambda e, b: (e, 0, 0)),
            pl.BlockSpec((TB, HIDDEN), lambda e, b: (b, 0)),
        ],
        out_specs=pl.BlockSpec(
            (TB, HIDDEN), lambda e, b: (jnp.where(e == E - 1, b, 0), 0)),
        out_shape=jax.ShapeDtypeStruct((T, HIDDEN), jnp.float32),
        scratch_shapes=[
            pltpu.VMEM((T, HIDDEN), jnp.float32),
            pltpu.VMEM((T, E), jnp.float32),
        ],
        compiler_params=pltpu.CompilerParams(
            dimension_semantics=("arbitrary", "arbitrary")),
    )(x, gate_w, Wg.astype(jnp.bfloat16), Wu.astype(jnp.bfloat16),
      Wd.astype(jnp.bfloat16), partial)

    return out.reshape(bsz, s, d)


# speedup vs baseline: 1.4638x; 1.2045x over previous
import functools

import jax
import jax.numpy as jnp
from jax import lax
from jax.experimental import pallas as pl
from jax.experimental.pallas import tpu as pltpu
from jax.experimental.pallas import tpu_sc as plsc

HIDDEN = 1024
MOE_FF = 512
SHARED_FF = 1024
E = 8
T = 2048
TK = 2 * T
TB = 256
NBMAX = 23
NSLOT = NBMAX * TB
TBS = 512

NW = 32
SPW = TK // NW
RCH = 64
TPW = T // NW


def _dot_t(a, b, prec=jnp.float32):
    return jax.lax.dot_general(a, b, (((1,), (1,)), ((), ())),
                               preferred_element_type=prec)


def _prefix_rows(m):
    acc = m
    sh = 1
    while sh < T:
        acc = acc + jnp.pad(acc[:, :T - sh], ((0, 0), (sh, 0)))
        sh *= 2
    return acc


def _router_body(x_ref, gw_ref, ti_ref, tw_ref, pos_ref, bexp_ref):
    lg = _dot_t(gw_ref[...], x_ref[...])
    iota = jax.lax.broadcasted_iota(jnp.int32, (E, T), 0)
    m1 = jnp.max(lg, axis=0, keepdims=True)
    i1 = jnp.min(jnp.where(lg == m1, iota, E), axis=0, keepdims=True)
    masked = jnp.where(iota == i1, -jnp.inf, lg)
    m2 = jnp.max(masked, axis=0, keepdims=True)
    i2 = jnp.min(jnp.where(masked == m2, iota, E), axis=0, keepdims=True)
    d = jnp.exp(m2 - m1)
    w1 = 1.0 / (1.0 + d)
    ti = jnp.concatenate([i1, i2], axis=0)
    ti_ref[...] = ti
    tw_ref[...] = jnp.concatenate([w1, 1.0 - w1], axis=0)

    rank = jnp.zeros((2, T), jnp.int32)
    cnt = jnp.zeros((1, E), jnp.int32)
    eiota = jax.lax.broadcasted_iota(jnp.int32, (1, E), 1)
    for e in range(E):
        me = jnp.where(ti == e, 1, 0)
        pre = _prefix_rows(me)
        tot0 = lax.slice(pre, (0, T - 1), (1, T))
        tot1 = lax.slice(pre, (1, T - 1), (2, T))
        carry = jnp.concatenate(
            [jnp.zeros((1, 1), jnp.int32), tot0], axis=0)
        re = pre - me + carry
        rank = rank + me * re
        cnt = cnt + jnp.where(eiota == e, tot0 + tot1, 0)

    nb = lax.shift_right_logical(cnt + (TB - 1), 8)
    blk = lax.shift_left(nb, 8)
    lt = (jax.lax.broadcasted_iota(jnp.int32, (E, E), 0)
          < jax.lax.broadcasted_iota(jnp.int32, (E, E), 1))
    excl = jax.lax.dot_general(
        blk.astype(jnp.float32), lt.astype(jnp.float32),
        (((1,), (0,)), ((), ())),
        preferred_element_type=jnp.float32).astype(jnp.int32)
    nbt = jnp.sum(nb, axis=1, keepdims=True)

    pos = rank
    for e in range(E):
        ex_e = lax.slice(excl, (0, e), (1, e + 1))
        pos = pos + jnp.where(ti == e, ex_e, 0)
    pos_ref[...] = pos

    biota = jax.lax.broadcasted_iota(jnp.int32, (1, 2 * LANES), 1)
    bb = jnp.minimum(biota, nbt - 1)
    acc = jnp.zeros((1, 2 * LANES), jnp.int32)
    exb = lax.shift_right_logical(excl, 8)
    for e in range(E):
        exb_e = lax.slice(exb, (0, e), (1, e + 1))
        acc = acc + jnp.where(bb >= exb_e, 1, 0)
    bexp_ref[...] = jnp.where(biota == NBMAX, nbt, acc - 1)


LANES = 16


def _dispatch_body(tok_hbm, pos_hbm, x_hbm, xs_hbm, idx_v, posv, rows_v, sem):
    wid = lax.axis_index("s") * 2 + lax.axis_index("c")
    base = wid * SPW
    for c in range(SPW // RCH):
        pltpu.sync_copy(tok_hbm.at[pl.ds(base + c * RCH, RCH)], idx_v)
        pltpu.sync_copy(pos_hbm.at[pl.ds(base + c * RCH, RCH)], posv)
        pltpu.async_copy(x_hbm.at[idx_v], rows_v, sem).wait()
        pltpu.async_copy(rows_v, xs_hbm.at[posv], sem).wait()


def _ffn_body(bexp_ref, xs_ref, wg_ref, wu_ref, wd_ref, ys_ref):
    i = pl.program_id(0)

    @pl.when(i < bexp_ref[NBMAX])
    def _do():
        xb = xs_ref[...].astype(jnp.bfloat16)
        g = _dot_t(xb, wg_ref[0])
        u = _dot_t(xb, wu_ref[0])
        h = (g * jax.nn.sigmoid(g) * u).astype(jnp.bfloat16)
        ys_ref[...] = _dot_t(h, wd_ref[0])


def _gather2_body(pos_hbm, ys_hbm, y0_hbm, y1_hbm, idx_v, rows_v, sem):
    wid = lax.axis_index("s") * 2 + lax.axis_index("c")
    tbase = wid * TPW
    pltpu.sync_copy(pos_hbm.at[pl.ds(tbase, TPW)], idx_v)
    pltpu.async_copy(ys_hbm.at[idx_v], rows_v, sem).wait()
    pltpu.sync_copy(rows_v, y0_hbm.at[pl.ds(tbase, TPW), :])
    pltpu.sync_copy(pos_hbm.at[pl.ds(T + tbase, TPW)], idx_v)
    pltpu.async_copy(ys_hbm.at[idx_v], rows_v, sem).wait()
    pltpu.sync_copy(rows_v, y1_hbm.at[pl.ds(tbase, TPW), :])


def _shared_combine_body(x_ref, sg_ref, su_ref, sd_ref, segw_ref,
                         y0_ref, y1_ref, w0_ref, w1_ref, out_ref):
    x = x_ref[...]
    xb = x.astype(jnp.bfloat16)
    g = _dot_t(xb, sg_ref[...])
    u = _dot_t(xb, su_ref[...])
    h = (g * jax.nn.sigmoid(g) * u).astype(jnp.bfloat16)
    sh = _dot_t(h, sd_ref[...])
    sgate = jax.nn.sigmoid(_dot_t(x, segw_ref[...]))
    out_ref[...] = (sgate * sh + w0_ref[...] * y0_ref[...]
                    + w1_ref[...] * y1_ref[...])


_SC_MESH = plsc.VectorSubcoreMesh(core_axis_name="c", subcore_axis_name="s",
                                  num_cores=2, num_subcores=16)

_dispatch = functools.partial(
    pl.kernel,
    mesh=_SC_MESH,
    compiler_params=pltpu.CompilerParams(needs_layout_passes=False),
    out_type=jax.ShapeDtypeStruct((NSLOT, HIDDEN), jnp.float32),
    scratch_types=[
        pltpu.VMEM((RCH,), jnp.int32),
        pltpu.VMEM((RCH,), jnp.int32),
        pltpu.VMEM((RCH, HIDDEN), jnp.float32),
        pltpu.SemaphoreType.DMA,
    ],
)(_dispatch_body)

_gather2 = functools.partial(
    pl.kernel,
    mesh=_SC_MESH,
    compiler_params=pltpu.CompilerParams(needs_layout_passes=False),
    out_type=[
        jax.ShapeDtypeStruct((T, HIDDEN), jnp.float32),
        jax.ShapeDtypeStruct((T, HIDDEN), jnp.float32),
    ],
    scratch_types=[
        pltpu.VMEM((TPW,), jnp.int32),
        pltpu.VMEM((TPW, HIDDEN), jnp.float32),
        pltpu.SemaphoreType.DMA,
    ],
)(_gather2_body)


@jax.jit
def kernel(hidden_states, gate_w, Wg, Wu, Wd, Sg, Su, Sd, seg_w):
    bsz, s, d = hidden_states.shape
    x = hidden_states.reshape(bsz * s, d)

    ti, tw, pos, bexp = pl.pallas_call(
        _router_body,
        in_specs=[
            pl.BlockSpec((T, HIDDEN), lambda: (0, 0)),
            pl.BlockSpec((E, HIDDEN), lambda: (0, 0)),
        ],
        out_specs=[
            pl.BlockSpec((2, T), lambda: (0, 0)),
            pl.BlockSpec((2, T), lambda: (0, 0)),
            pl.BlockSpec((2, T), lambda: (0, 0)),
            pl.BlockSpec((1, 2 * LANES), lambda: (0, 0)),
        ],
        out_shape=[
            jax.ShapeDtypeStruct((2, T), jnp.int32),
            jax.ShapeDtypeStruct((2, T), jnp.float32),
            jax.ShapeDtypeStruct((2, T), jnp.int32),
            jax.ShapeDtypeStruct((1, 2 * LANES), jnp.int32),
        ],
    )(x, gate_w)

    tok_ids = jnp.tile(jnp.arange(T, dtype=jnp.int32), 2)
    xs = _dispatch(tok_ids, pos.reshape(TK), x)

    ys = pl.pallas_call(
        _ffn_body,
        grid_spec=pltpu.PrefetchScalarGridSpec(
            num_scalar_prefetch=1,
            grid=(NBMAX,),
            in_specs=[
                pl.BlockSpec((TB, HIDDEN),
                             lambda i, be: (jnp.minimum(i, be[NBMAX] - 1), 0)),
                pl.BlockSpec((1, MOE_FF, HIDDEN), lambda i, be: (be[i], 0, 0)),
                pl.BlockSpec((1, MOE_FF, HIDDEN), lambda i, be: (be[i], 0, 0)),
                pl.BlockSpec((1, HIDDEN, MOE_FF), lambda i, be: (be[i], 0, 0)),
            ],
            out_specs=pl.BlockSpec(
                (TB, HIDDEN), lambda i, be: (jnp.minimum(i, be[NBMAX] - 1), 0)),
        ),
        out_shape=jax.ShapeDtypeStruct((NSLOT, HIDDEN), jnp.float32),
        compiler_params=pltpu.CompilerParams(
            dimension_semantics=("arbitrary",)),
    )(bexp.reshape(2 * LANES), xs, Wg.astype(jnp.bfloat16),
      Wu.astype(jnp.bfloat16), Wd.astype(jnp.bfloat16))

    y0, y1 = _gather2(pos.reshape(TK), ys)

    out = pl.pallas_call(
        _shared_combine_body,
        grid=(T // TBS,),
        in_specs=[
            pl.BlockSpec((TBS, HIDDEN), lambda i: (i, 0)),
            pl.BlockSpec((SHARED_FF, HIDDEN), lambda i: (0, 0)),
            pl.BlockSpec((SHARED_FF, HIDDEN), lambda i: (0, 0)),
            pl.BlockSpec((HIDDEN, SHARED_FF), lambda i: (0, 0)),
            pl.BlockSpec((1, HIDDEN), lambda i: (0, 0)),
            pl.BlockSpec((TBS, HIDDEN), lambda i: (i, 0)),
            pl.BlockSpec((TBS, HIDDEN), lambda i: (i, 0)),
            pl.BlockSpec((TBS, 1), lambda i: (i, 0)),
            pl.BlockSpec((TBS, 1), lambda i: (i, 0)),
        ],
        out_specs=pl.BlockSpec((TBS, HIDDEN), lambda i: (i, 0)),
        out_shape=jax.ShapeDtypeStruct((T, HIDDEN), jnp.float32),
    )(x, Sg.astype(jnp.bfloat16), Su.astype(jnp.bfloat16),
      Sd.astype(jnp.bfloat16), seg_w, y0, y1,
      tw[0].reshape(T, 1), tw[1].reshape(T, 1))

    return out.reshape(bsz, s, d)
